# Initial kernel scaffold; baseline (speedup 1.0000x reference)
import jax
import jax.numpy as jnp
from jax.experimental import pallas as pl


def kernel(indices, entity_types, entity_data_idx, tables, W, b, lookup_table):
    raise NotImplementedError("write your pallas kernel here")


# speedup vs baseline: 3.6688x; 3.6688x over previous
import functools

import jax
import jax.numpy as jnp
from jax import lax
from jax.experimental import pallas as pl
from jax.experimental.pallas import tpu as pltpu
from jax.experimental.pallas import tpu_sc as plsc

NUM_TYPES = 8
DATA_SIZE = 10000
IN_DIM = 256
DIM = 128
BATCH = 16384

NUM_WORKERS = 32
PER_WORKER = BATCH // NUM_WORKERS
CHUNK = 128
NUM_CHUNKS = PER_WORKER // CHUNK

TC_BLOCK = 512
NUM_TC_BLOCKS = BATCH // TC_BLOCK


def _sc_gather(codes, indices, tables_flat, lookup_table):
    mesh = plsc.VectorSubcoreMesh(core_axis_name="c", subcore_axis_name="s")

    @functools.partial(
        pl.kernel,
        out_type=(
            jax.ShapeDtypeStruct((BATCH, IN_DIM), jnp.float32),
            jax.ShapeDtypeStruct((BATCH, DIM), jnp.float32),
            jax.ShapeDtypeStruct((BATCH,), jnp.int32),
        ),
        mesh=mesh,
        scratch_types=[
            pltpu.VMEM((PER_WORKER,), jnp.int32),
            pltpu.VMEM((CHUNK,), jnp.int32),
            pltpu.VMEM((CHUNK,), jnp.int32),
            pltpu.VMEM((CHUNK, IN_DIM), jnp.float32),
            pltpu.VMEM((CHUNK, DIM), jnp.float32),
            pltpu.SemaphoreType.DMA,
        ],
    )
    def sc_kernel(codes_hbm, idx_hbm, tab_hbm, lut_hbm,
                  emb_out, lb_out, code_out,
                  idx_v, c_v, row_v, emb_v, lb_v, sem):
        wid = lax.axis_index("s") * 2 + lax.axis_index("c")
        base = wid * PER_WORKER
        pltpu.sync_copy(idx_hbm.at[pl.ds(base, PER_WORKER)], idx_v)
        for k in range(NUM_CHUNKS):
            off = k * CHUNK
            idx_slice = idx_v.at[pl.ds(off, CHUNK)]
            pltpu.async_copy(codes_hbm.at[idx_slice], c_v, sem).wait()

            @pl.loop(0, CHUNK, step=16)
            def _(i):
                c = c_v[pl.ds(i, 16)]
                row_v[pl.ds(i, 16)] = jnp.maximum(c, 0)

            pltpu.async_copy(tab_hbm.at[row_v], emb_v, sem).wait()
            pltpu.async_copy(lut_hbm.at[idx_slice], lb_v, sem).wait()
            pltpu.sync_copy(c_v, code_out.at[pl.ds(base + off, CHUNK)])
            pltpu.sync_copy(emb_v, emb_out.at[pl.ds(base + off, CHUNK)])
            pltpu.sync_copy(lb_v, lb_out.at[pl.ds(base + off, CHUNK)])

    return sc_kernel(codes, indices, tables_flat, lookup_table)


def _tc_body(c_ref, emb_ref, lb_ref, w_ref, b_ref, o_ref):
    c = c_ref[...]
    emb = emb_ref[...]
    p = jnp.dot(emb, w_ref[...], preferred_element_type=jnp.float32)
    t = c // DATA_SIZE
    acc = jnp.zeros((TC_BLOCK, DIM), jnp.float32)
    for tt in range(NUM_TYPES):
        seg = p[:, tt * DIM:(tt + 1) * DIM] + b_ref[tt, :][None, :]
        acc = acc + jnp.where(t == tt, seg, 0.0)
    o_ref[...] = jnp.where(c >= 0, acc, lb_ref[...])


def _tc_combine(codes_b, emb, lb, w_wide, b):
    return pl.pallas_call(
        _tc_body,
        grid=(NUM_TC_BLOCKS,),
        in_specs=[
            pl.BlockSpec((TC_BLOCK, 1), lambda i: (i, 0)),
            pl.BlockSpec((TC_BLOCK, IN_DIM), lambda i: (i, 0)),
            pl.BlockSpec((TC_BLOCK, DIM), lambda i: (i, 0)),
            pl.BlockSpec((IN_DIM, NUM_TYPES * DIM), lambda i: (0, 0)),
            pl.BlockSpec((NUM_TYPES, DIM), lambda i: (0, 0)),
        ],
        out_specs=pl.BlockSpec((TC_BLOCK, DIM), lambda i: (i, 0)),
        out_shape=jax.ShapeDtypeStruct((BATCH, DIM), jnp.float32),
        compiler_params=pltpu.CompilerParams(
            dimension_semantics=("arbitrary",),
        ),
    )(codes_b, emb, lb, w_wide, b)


def kernel(indices, entity_types, entity_data_idx, tables, W, b, lookup_table):
    indices = indices.astype(jnp.int32)
    entity_types = entity_types.astype(jnp.int32)
    entity_data_idx = entity_data_idx.astype(jnp.int32)
    codes = jnp.where(entity_types < NUM_TYPES,
                      entity_types * DATA_SIZE + entity_data_idx,
                      -1).astype(jnp.int32)
    tables_flat = tables.reshape(NUM_TYPES * DATA_SIZE, IN_DIM)
    w_wide = jnp.transpose(W, (1, 0, 2)).reshape(IN_DIM, NUM_TYPES * DIM)

    emb, lb, codes_b = _sc_gather(codes, indices, tables_flat, lookup_table)
    return _tc_combine(codes_b.reshape(BATCH, 1), emb, lb, w_wide, b)
